# baseline (speedup 1.0000x reference)
import functools

import jax
import jax.numpy as jnp
from jax import lax
from jax.experimental import pallas as pl
from jax.experimental.pallas import tpu as pltpu
from jax.experimental.pallas import tpu_sc as plsc

B = 16384
ED = 128
MD = 8
MW = 4 * MD
MROWS = 130
ROWOFF = (0, 2, 9, 30)

_info = plsc.get_sparse_core_info()
NC, NS = _info.num_cores, _info.num_subcores
NW = NC * NS
BPW = B // NW
CH = 128
NCH = BPW // CH
NBUF = 2


def _sc_gather(uid2, mid2, g2, a2, o2, z2, uemb, memb, mtab):
    mesh = plsc.VectorSubcoreMesh(core_axis_name="c", subcore_axis_name="s")

    idx2 = lambda: pltpu.VMEM((NCH, CH), jnp.int32)
    rowbuf = lambda: pltpu.VMEM((CH, ED), jnp.float32)

    @functools.partial(
        pl.kernel,
        mesh=mesh,
        compiler_params=pltpu.CompilerParams(needs_layout_passes=False),
        out_type=[
            jax.ShapeDtypeStruct((B, ED), jnp.float32),
            jax.ShapeDtypeStruct((B, ED), jnp.float32),
            jax.ShapeDtypeStruct((B, MW), jnp.float32),
        ],
        scratch_types=(
            [idx2() for _ in range(6)]
            + [rowbuf() for _ in range(2 * NBUF)]
            + [pltpu.VMEM((MROWS, MD), jnp.float32)]
            + [pltpu.VMEM((CH, MW), jnp.float32) for _ in range(2)]
            + [pltpu.SemaphoreType.DMA for _ in range(11)]
        ),
    )
    def body(uid_h, mid_h, g_h, a_h, o_h, z_h, uemb_h, memb_h, mtab_h,
             ulat_h, mlat_h, meta_h,
             uix, mix, gix, aix, oix, zix,
             ub0, ub1, mb0, mb1, mt_v, ms0, ms1,
             s_init, sg_u0, sg_u1, sg_m0, sg_m1,
             st_u0, st_u1, st_m0, st_m1, st_ms0, st_ms1):
        ub = (ub0, ub1)
        mb = (mb0, mb1)
        ms = (ms0, ms1)
        sg_u = (sg_u0, sg_u1)
        sg_m = (sg_m0, sg_m1)
        st_u = (st_u0, st_u1)
        st_m = (st_m0, st_m1)
        st_ms = (st_ms0, st_ms1)

        wid = lax.axis_index("s") * NC + lax.axis_index("c")
        base = wid * BPW

        inits = [
            pltpu.async_copy(uid_h.at[pl.ds(wid * NCH, NCH)], uix, s_init),
            pltpu.async_copy(mid_h.at[pl.ds(wid * NCH, NCH)], mix, s_init),
            pltpu.async_copy(g_h.at[pl.ds(wid * NCH, NCH)], gix, s_init),
            pltpu.async_copy(a_h.at[pl.ds(wid * NCH, NCH)], aix, s_init),
            pltpu.async_copy(o_h.at[pl.ds(wid * NCH, NCH)], oix, s_init),
            pltpu.async_copy(z_h.at[pl.ds(wid * NCH, NCH)], zix, s_init),
            pltpu.async_copy(mtab_h, mt_v, s_init),
        ]
        for cp in inits:
            cp.wait()

        lane = lax.broadcasted_iota(jnp.int32, (16,), 0)

        def meta_compute(c, msbuf):
            for s in range(CH // 16):
                row16 = lane + s * 16
                for t, (tix, roff) in enumerate(
                        zip((gix, aix, oix, zix), ROWOFF)):
                    id16 = tix[c, pl.ds(s * 16, 16)] + roff
                    for j in range(MD):
                        colj = jnp.full((16,), j, jnp.int32)
                        v = plsc.load_gather(mt_v, [id16, colj])
                        col16 = jnp.full((16,), t * MD + j, jnp.int32)
                        plsc.store_scatter(msbuf, [row16, col16], v)

        gu = [None] * NBUF
        gm = [None] * NBUF
        for c in range(NBUF):
            gu[c] = pltpu.async_copy(uemb_h.at[uix.at[c]], ub[c], sg_u[c])
            gm[c] = pltpu.async_copy(memb_h.at[mix.at[c]], mb[c], sg_m[c])

        stu = [None] * NCH
        stm = [None] * NCH
        stms = [None] * NCH
        for c in range(NCH):
            sl = c % NBUF
            rows = pl.ds(base + c * CH, CH)
            gu[sl].wait()
            gm[sl].wait()
            stu[c] = pltpu.async_copy(ub[sl], ulat_h.at[rows], st_u[sl])
            stm[c] = pltpu.async_copy(mb[sl], mlat_h.at[rows], st_m[sl])
            msl = c % 2
            if c >= 2:
                stms[c - 2].wait()
            meta_compute(c, ms[msl])
            stms[c] = pltpu.async_copy(ms[msl], meta_h.at[rows], st_ms[msl])
            nxt = c + NBUF
            if nxt < NCH:
                stu[c].wait()
                stm[c].wait()
                gu[sl] = pltpu.async_copy(uemb_h.at[uix.at[nxt]], ub[sl],
                                          sg_u[sl])
                gm[sl] = pltpu.async_copy(memb_h.at[mix.at[nxt]], mb[sl],
                                          sg_m[sl])
        for c in range(NCH):
            if stu[c] is not None and c + NBUF >= NCH:
                stu[c].wait()
                stm[c].wait()
            if c >= NCH - 2:
                stms[c].wait()

    return body(uid2, mid2, g2, a2, o2, z2, uemb, memb, mtab)


BLK = 1024


def _tc_body(u_ref, m_ref, mt_ref, wu_ref, wm_ref, b_ref, out_ref):
    t = jnp.dot(u_ref[...], wu_ref[...], preferred_element_type=jnp.float32)
    t += jnp.dot(mt_ref[...], wm_ref[...], preferred_element_type=jnp.float32)
    t += b_ref[...]
    out_ref[...] = jnp.sum(t * m_ref[...], axis=1)


def _tc_call(ulat, mlat, meta, wu, wm, bb):
    grid = (B // BLK,)
    row = lambda i: (i, 0)
    rep = lambda i: (0, 0)
    return pl.pallas_call(
        _tc_body,
        grid=grid,
        in_specs=[
            pl.BlockSpec((BLK, ED), row),
            pl.BlockSpec((BLK, ED), row),
            pl.BlockSpec((BLK, MW), row),
            pl.BlockSpec((ED, ED), rep),
            pl.BlockSpec((MW, ED), rep),
            pl.BlockSpec((1, ED), rep),
        ],
        out_specs=pl.BlockSpec((BLK,), lambda i: (i,)),
        out_shape=jax.ShapeDtypeStruct((B,), jnp.float32),
    )(ulat, mlat, meta, wu, wm, bb)


def kernel(user_id, movie_id, gender, age, occupation, zip_code,
           user_emb, movie_emb, gender_emb, age_emb, occupation_emb, zip_emb,
           W, b):
    mtab = jnp.concatenate(
        [gender_emb, age_emb, occupation_emb, zip_emb], axis=0)
    wu = W[:ED]
    wm = W[ED:]
    bb = b.reshape(1, ED)
    r2 = lambda x: x.reshape(NW * NCH, CH)

    ulat, mlat, meta = _sc_gather(
        r2(user_id), r2(movie_id), r2(gender), r2(age),
        r2(occupation), r2(zip_code), user_emb, movie_emb, mtab)
    return _tc_call(ulat, mlat, meta, wu, wm, bb)


# speedup vs baseline: 1.4107x; 1.4107x over previous
import functools

import jax
import jax.numpy as jnp
from jax import lax
from jax.experimental import pallas as pl
from jax.experimental.pallas import tpu as pltpu
from jax.experimental.pallas import tpu_sc as plsc

B = 16384
ED = 128
MD = 8
MW = 4 * MD
MTOT = 1040
ELOFF = (0, 16, 72, 240)

_info = plsc.get_sparse_core_info()
NC, NS = _info.num_cores, _info.num_subcores
NW = NC * NS
BPW = B // NW
CH = 128
NCH = BPW // CH
NBUF = 2


def _sc_gather(uid2, mid2, g2, a2, o2, z2, uemb, memb, mtab):
    mesh = plsc.VectorSubcoreMesh(core_axis_name="c", subcore_axis_name="s")

    idx2 = lambda: pltpu.VMEM((NCH, CH), jnp.int32)
    rowbuf = lambda: pltpu.VMEM((CH, ED), jnp.float32)

    @functools.partial(
        pl.kernel,
        mesh=mesh,
        compiler_params=pltpu.CompilerParams(needs_layout_passes=False),
        out_type=[
            jax.ShapeDtypeStruct((B, ED), jnp.float32),
            jax.ShapeDtypeStruct((B, ED), jnp.float32),
            jax.ShapeDtypeStruct((MW, B), jnp.float32),
        ],
        scratch_types=(
            [idx2() for _ in range(6)]
            + [rowbuf() for _ in range(2 * NBUF)]
            + [pltpu.VMEM((MTOT,), jnp.float32)]
            + [pltpu.VMEM((MW, BPW), jnp.float32)]
            + [pltpu.SemaphoreType.DMA for _ in range(10)]
        ),
    )
    def body(uid_h, mid_h, g_h, a_h, o_h, z_h, uemb_h, memb_h, mtab_h,
             ulat_h, mlat_h, meta_h,
             uix, mix, gix, aix, oix, zix,
             ub0, ub1, mb0, mb1, mt_v, ms_v,
             s_init, sg_u0, sg_u1, sg_m0, sg_m1,
             st_u0, st_u1, st_m0, st_m1, st_ms):
        ub = (ub0, ub1)
        mb = (mb0, mb1)
        sg_u = (sg_u0, sg_u1)
        sg_m = (sg_m0, sg_m1)
        st_u = (st_u0, st_u1)
        st_m = (st_m0, st_m1)

        wid = lax.axis_index("s") * NC + lax.axis_index("c")
        base = wid * BPW

        inits = [
            pltpu.async_copy(uid_h.at[pl.ds(wid * NCH, NCH)], uix, s_init),
            pltpu.async_copy(mid_h.at[pl.ds(wid * NCH, NCH)], mix, s_init),
            pltpu.async_copy(g_h.at[pl.ds(wid * NCH, NCH)], gix, s_init),
            pltpu.async_copy(a_h.at[pl.ds(wid * NCH, NCH)], aix, s_init),
            pltpu.async_copy(o_h.at[pl.ds(wid * NCH, NCH)], oix, s_init),
            pltpu.async_copy(z_h.at[pl.ds(wid * NCH, NCH)], zix, s_init),
            pltpu.async_copy(mtab_h, mt_v, s_init),
        ]
        for cp in inits:
            cp.wait()

        gu = [None] * NBUF
        gm = [None] * NBUF
        for c in range(NBUF):
            gu[c] = pltpu.async_copy(uemb_h.at[uix.at[c]], ub[c], sg_u[c])
            gm[c] = pltpu.async_copy(memb_h.at[mix.at[c]], mb[c], sg_m[c])

        for c in range(NCH):
            for s in range(CH // 16):
                pos = pl.ds(c * CH + s * 16, 16)
                for t, (tix, eoff) in enumerate(
                        zip((gix, aix, oix, zix), ELOFF)):
                    fb = tix[c, pl.ds(s * 16, 16)] * MD + eoff
                    for j in range(MD):
                        ms_v[t * MD + j, pos] = plsc.load_gather(
                            mt_v, [fb + j])
        stms = pltpu.async_copy(ms_v, meta_h.at[:, pl.ds(base, BPW)], st_ms)

        stu = [None] * NCH
        stm = [None] * NCH
        for c in range(NCH):
            sl = c % NBUF
            rows = pl.ds(base + c * CH, CH)
            gu[sl].wait()
            gm[sl].wait()
            stu[c] = pltpu.async_copy(ub[sl], ulat_h.at[rows], st_u[sl])
            stm[c] = pltpu.async_copy(mb[sl], mlat_h.at[rows], st_m[sl])
            nxt = c + NBUF
            if nxt < NCH:
                stu[c].wait()
                stm[c].wait()
                gu[sl] = pltpu.async_copy(uemb_h.at[uix.at[nxt]], ub[sl],
                                          sg_u[sl])
                gm[sl] = pltpu.async_copy(memb_h.at[mix.at[nxt]], mb[sl],
                                          sg_m[sl])
        for c in range(NCH - NBUF, NCH):
            stu[c].wait()
            stm[c].wait()
        stms.wait()

    return body(uid2, mid2, g2, a2, o2, z2, uemb, memb, mtab)


BLK = 2048


def _tc_body(u_ref, m_ref, mt_ref, wu_ref, wm_ref, b_ref, out_ref):
    t = jnp.dot(u_ref[...], wu_ref[...], preferred_element_type=jnp.float32)
    t += lax.dot_general(mt_ref[...], wm_ref[...],
                         (((0,), (0,)), ((), ())),
                         preferred_element_type=jnp.float32)
    t += b_ref[...]
    p = t * m_ref[...]
    ones8 = jnp.ones((8, ED), jnp.float32)
    o8 = lax.dot_general(ones8, p, (((1,), (1,)), ((), ())),
                         preferred_element_type=jnp.float32)
    out_ref[...] = o8[0:1, :].reshape(1, 1, BLK)


def _tc_call(ulat, mlat, meta, wu, wm, bb):
    grid = (B // BLK,)
    row = lambda i: (i, 0)
    rep = lambda i: (0, 0)
    return pl.pallas_call(
        _tc_body,
        grid=grid,
        in_specs=[
            pl.BlockSpec((BLK, ED), row),
            pl.BlockSpec((BLK, ED), row),
            pl.BlockSpec((MW, BLK), lambda i: (0, i)),
            pl.BlockSpec((ED, ED), rep),
            pl.BlockSpec((MW, ED), rep),
            pl.BlockSpec((1, ED), rep),
        ],
        out_specs=pl.BlockSpec((1, 1, BLK), lambda i: (i, 0, 0)),
        out_shape=jax.ShapeDtypeStruct((B // BLK, 1, BLK), jnp.float32),
    )(ulat, mlat, meta, wu, wm, bb).reshape(B)


def kernel(user_id, movie_id, gender, age, occupation, zip_code,
           user_emb, movie_emb, gender_emb, age_emb, occupation_emb, zip_emb,
           W, b):
    mtab = jnp.concatenate(
        [gender_emb.reshape(-1), age_emb.reshape(-1),
         occupation_emb.reshape(-1), zip_emb.reshape(-1)])
    wu = W[:ED]
    wm = W[ED:]
    bb = b.reshape(1, ED)
    r2 = lambda x: x.reshape(NW * NCH, CH)

    ulat, mlat, meta = _sc_gather(
        r2(user_id), r2(movie_id), r2(gender), r2(age),
        r2(occupation), r2(zip_code), user_emb, movie_emb, mtab)
    return _tc_call(ulat, mlat, meta, wu, wm, bb)
